# baseline (speedup 1.0000x reference)
import jax
import jax.numpy as jnp
from jax.experimental import pallas as pl

K = 5000


def _copy_body(i_ref, o_ref):
    o_ref[...] = i_ref[...]


def kernel(x):
    vals, idx = jax.lax.top_k(x, K)
    order = jnp.argsort(idx)
    out = vals[order]
    return pl.pallas_call(
        _copy_body,
        out_shape=jax.ShapeDtypeStruct((K,), jnp.float32),
    )(out)


# speedup vs baseline: 7.0836x; 7.0836x over previous
import functools

import jax
import jax.numpy as jnp
from jax import lax
from jax.experimental import pallas as pl
from jax.experimental.pallas import tpu as pltpu
from jax.experimental.pallas import tpu_sc as plsc

K = 5000
N = 1000000
NW = 32
CHUNK = 31264
LAST = N - 31 * CHUNK
NBIN = 4096
SPAN = NBIN // 16
CAP = 512
NCAND = NW * CAP

MESH = plsc.VectorSubcoreMesh(core_axis_name="c", subcore_axis_name="s")
CP = pltpu.CompilerParams(needs_layout_passes=False)


def _keys(w):
    u = lax.bitcast_convert_type(w, jnp.int32)
    return u ^ lax.shift_right_logical(lax.shift_right_arithmetic(u, 31), 1)


def _load_chunk(x_hbm, chunk, wid):
    base = wid * CHUNK

    @pl.when(wid < 31)
    def _():
        pltpu.sync_copy(x_hbm.at[pl.ds(base, CHUNK)], chunk)

    @pl.when(wid == 31)
    def _():
        pltpu.sync_copy(x_hbm.at[pl.ds(base, LAST)], chunk.at[pl.ds(0, LAST)])

    return jnp.where(wid == 31, LAST // 16, CHUNK // 16)


def _hist_accum(hist, bins):
    cnt, last = plsc.scan_count(bins)
    cur = plsc.load_gather(hist, [bins], mask=last)
    plsc.store_scatter(hist, [bins], cur + cnt, mask=last)


def _sum_hist(hist2_hbm, hraw, hsum):
    pltpu.sync_copy(hist2_hbm, hraw)

    def body(i, _):
        hsum[pl.ds(i * 16, 16)] = (
            hraw[pl.ds(i * 16, 16)] + hraw[pl.ds(NBIN + i * 16, 16)]
        )
        return 0

    lax.fori_loop(0, NBIN // 16, body, 0)


def _scan_topbin(h_ref, nvb, kneed):
    iota16 = lax.iota(jnp.int32, 16)

    def body(j, st):
        carry, found, bsel, nab = st
        i = nvb - 1 - j
        v = h_ref[pl.ds(i * 16, 16)]
        sfx = lax.rev(plsc.cumsum(lax.rev(v, (0,))), (0,)) + carry
        cross = sfx >= kneed
        pc0 = plsc.all_reduce_population_count(cross)[0]
        hit = (found == 0) & (pc0 > 0)
        lane = pc0 - 1
        ngesel = jnp.sum(jnp.where(iota16 == lane, sfx, 0))
        hvsel = jnp.sum(jnp.where(iota16 == lane, v, 0))
        bsel = jnp.where(hit, i * 16 + lane, bsel)
        nab = jnp.where(hit, ngesel - hvsel, nab)
        found = jnp.where(hit, jnp.int32(1), found)
        return sfx[0], found, bsel, nab

    _, _, bsel, nab = lax.fori_loop(
        0, nvb, body,
        (jnp.int32(0), jnp.int32(0), jnp.int32(0), jnp.int32(0)))
    return bsel, kneed - nab


@functools.partial(
    pl.kernel, mesh=MESH, compiler_params=CP,
    out_type=jax.ShapeDtypeStruct((2 * NBIN,), jnp.int32),
    scratch_types=[
        pltpu.VMEM((CHUNK,), jnp.float32),
        pltpu.VMEM((NBIN,), jnp.int32),
        pltpu.VMEM((SPAN,), jnp.int32),
        pltpu.VMEM((SPAN,), jnp.int32),
        pltpu.VMEM_SHARED((16 * 16, SPAN), jnp.int32),
    ],
)
def _hist_kernel(x_hbm, out_hbm, chunk, hist, acc, tmp, srows):
    c = lax.axis_index("c")
    s = lax.axis_index("s")
    wid = s * 2 + c

    def zbody(i, _):
        hist[pl.ds(i * 16, 16)] = jnp.zeros((16,), jnp.int32)
        return 0

    lax.fori_loop(0, NBIN // 16, zbody, 0)

    nv = _load_chunk(x_hbm, chunk, wid)

    def body(i, _):
        ks = _keys(chunk[pl.ds(i * 16, 16)])
        bins = lax.shift_right_arithmetic(ks, 20) + 2048
        _hist_accum(hist, bins)
        return 0

    lax.fori_loop(0, nv, body, 0)

    def pub(k, _):
        pltpu.sync_copy(hist.at[pl.ds(k * SPAN, SPAN)], srows.at[s * 16 + k])
        return 0

    lax.fori_loop(0, 16, pub, 0)
    plsc.subcore_barrier()

    def zacc(i, _):
        acc[pl.ds(i * 16, 16)] = jnp.zeros((16,), jnp.int32)
        return 0

    lax.fori_loop(0, SPAN // 16, zacc, 0)

    def comb(r, _):
        pltpu.sync_copy(srows.at[r * 16 + s], tmp)

        def addv(i, _):
            acc[pl.ds(i * 16, 16)] = acc[pl.ds(i * 16, 16)] + tmp[pl.ds(i * 16, 16)]
            return 0

        lax.fori_loop(0, SPAN // 16, addv, 0)
        return 0

    lax.fori_loop(0, 16, comb, 0)
    pltpu.sync_copy(acc, out_hbm.at[pl.ds(c * NBIN + s * SPAN, SPAN)])


@functools.partial(
    pl.kernel, mesh=MESH, compiler_params=CP,
    out_type=jax.ShapeDtypeStruct((NCAND,), jnp.float32),
    scratch_types=[
        pltpu.VMEM((CHUNK,), jnp.float32),
        pltpu.VMEM((2 * NBIN,), jnp.int32),
        pltpu.VMEM((NBIN,), jnp.int32),
        pltpu.VMEM((CAP,), jnp.float32),
    ],
)
def _cand_kernel(x_hbm, hist2_hbm, out_hbm, chunk, hraw, hsum, cand):
    c = lax.axis_index("c")
    s = lax.axis_index("s")
    wid = s * 2 + c

    _sum_hist(hist2_hbm, hraw, hsum)
    b1, _ = _scan_topbin(hsum, NBIN // 16, jnp.int32(K))
    lo1 = lax.shift_left(b1 - 2048, 20)

    sent = lax.bitcast_convert_type(jnp.full((16,), -1, jnp.int32), jnp.float32)

    def zbody(i, _):
        cand[pl.ds(i * 16, 16)] = sent
        return 0

    lax.fori_loop(0, CAP // 16, zbody, 0)

    nv = _load_chunk(x_hbm, chunk, wid)

    def body(i, off):
        w = chunk[pl.ds(i * 16, 16)]
        ks = _keys(w)
        sel = ks >= lo1
        seli = jnp.where(sel, jnp.int32(1), jnp.int32(0))
        pos = off + plsc.cumsum(seli) - seli
        pos = jnp.minimum(pos, CAP - 1)
        plsc.store_scatter(cand, [jnp.where(sel, pos, 0)], w, mask=sel)
        return off + plsc.all_reduce_population_count(sel)

    lax.fori_loop(0, nv, body, jnp.zeros((16,), jnp.int32))
    pltpu.sync_copy(cand, out_hbm.at[pl.ds(wid * CAP, CAP)])


@functools.partial(
    pl.kernel, mesh=MESH, compiler_params=CP,
    out_type=jax.ShapeDtypeStruct((K,), jnp.float32),
    scratch_types=[
        pltpu.VMEM((NCAND,), jnp.float32),
        pltpu.VMEM((NCAND,), jnp.int32),
        pltpu.VMEM((2 * NBIN,), jnp.int32),
        pltpu.VMEM((NBIN,), jnp.int32),
        pltpu.VMEM((NBIN + 16,), jnp.int32),
        pltpu.VMEM((256 + 16,), jnp.int32),
        pltpu.VMEM((K + 16,), jnp.float32),
    ],
)
def _select_kernel(cand_hbm, hist2_hbm, out_hbm, cv, ck, hraw, hsum, h2, h3, outv):
    c = lax.axis_index("c")
    s = lax.axis_index("s")

    @pl.when((c == 0) & (s == 0))
    def _():
        _sum_hist(hist2_hbm, hraw, hsum)
        b1, k1 = _scan_topbin(hsum, NBIN // 16, jnp.int32(K))

        def z2(i, _):
            h2[pl.ds(i * 16, 16)] = jnp.zeros((16,), jnp.int32)
            return 0

        lax.fori_loop(0, (NBIN + 16) // 16, z2, 0)

        def z3(i, _):
            h3[pl.ds(i * 16, 16)] = jnp.zeros((16,), jnp.int32)
            return 0

        lax.fori_loop(0, (256 + 16) // 16, z3, 0)

        pltpu.sync_copy(cand_hbm, cv)
        nvc = NCAND // 16
        top1 = b1 - 2048
        smin = jnp.int32(-(2 ** 31))

        def pa(i, _):
            ks = _keys(cv[pl.ds(i * 16, 16)])
            ck[pl.ds(i * 16, 16)] = ks
            m1 = (lax.shift_right_arithmetic(ks, 20) == top1) & (ks != smin)
            bins = jnp.where(
                m1, lax.shift_right_arithmetic(ks, 8) & 0xFFF, jnp.int32(NBIN))
            _hist_accum(h2, bins)
            return 0

        lax.fori_loop(0, nvc, pa, 0)
        b2, k2 = _scan_topbin(h2, NBIN // 16, k1)
        hi20 = lax.shift_left(top1, 12) + b2

        def pb(i, _):
            ks = ck[pl.ds(i * 16, 16)]
            m2 = (lax.shift_right_arithmetic(ks, 8) == hi20) & (ks != smin)
            bins = jnp.where(m2, ks & 0xFF, jnp.int32(256))
            _hist_accum(h3, bins)
            return 0

        lax.fori_loop(0, nvc, pb, 0)
        b3, k3 = _scan_topbin(h3, 256 // 16, k2)
        t_key = lax.shift_left(hi20, 8) + b3

        k3v = jnp.zeros((16,), jnp.int32) + k3

        def pc(i, st):
            off, eqrun = st
            sl = pl.ds(i * 16, 16)
            ks = ck[sl]
            w = cv[sl]
            gt = ks > t_key
            eq = ks == t_key
            eqi = jnp.where(eq, jnp.int32(1), jnp.int32(0))
            eq_excl = eqrun + plsc.cumsum(eqi) - eqi
            take = gt | (eq & (eq_excl < k3v))
            ti = jnp.where(take, jnp.int32(1), jnp.int32(0))
            pos = off + plsc.cumsum(ti) - ti
            pos = jnp.minimum(pos, K + 15)
            plsc.store_scatter(outv, [jnp.where(take, pos, 0)], w, mask=take)
            return (off + plsc.all_reduce_population_count(take),
                    eqrun + plsc.all_reduce_population_count(eq))

        z16 = jnp.zeros((16,), jnp.int32)
        lax.fori_loop(0, nvc, pc, (z16, z16))
        pltpu.sync_copy(outv.at[pl.ds(0, K)], out_hbm)


def kernel(x):
    hist = _hist_kernel(x)
    cand = _cand_kernel(x, hist)
    return _select_kernel(cand, hist)
